# Initial kernel scaffold; baseline (speedup 1.0000x reference)
import jax
import jax.numpy as jnp
from jax.experimental import pallas as pl


def kernel(x, edge_index, observation, W_gat, att_src, att_dst, b_gat, aW1, ab1, aW2, ab2, aW3, ab3, cW1, cb1, cW2, cb2, cW3, cb3):
    raise NotImplementedError("write your pallas kernel here")


# speedup vs baseline: 98.3711x; 98.3711x over previous
import functools

import jax
import jax.numpy as jnp
from jax import lax
from jax.experimental import pallas as pl
from jax.experimental.pallas import tpu as pltpu
from jax.experimental.pallas import tpu_sc as plsc

NC = 2
NS = 16
NW = NC * NS
CH = 128
ZW = 32
TEAM_MOD = 10
TEAM_KEEP = 5


def _prologue_body(H, C, x_ref, w_ref, ats_ref, atd_ref,
                   asrc_ref, adst_ref, m_ref, ms_ref, md_ref):
    i = pl.program_id(0)
    nb = pl.num_programs(0)
    blk = x_ref.shape[0]
    h = jnp.dot(x_ref[...], w_ref[...], preferred_element_type=jnp.float32)
    h3 = h.reshape(blk, H, C)
    a_s = jnp.sum(h3 * ats_ref[...][None], axis=-1)
    a_d = jnp.sum(h3 * atd_ref[...][None], axis=-1)
    asrc_ref[...] = a_s
    adst_ref[...] = a_d
    bs = jnp.max(a_s, axis=0, keepdims=True)
    bd = jnp.max(a_d, axis=0, keepdims=True)

    @pl.when(i == 0)
    def _():
        ms_ref[...] = bs
        md_ref[...] = bd

    @pl.when(i > 0)
    def _():
        ms_ref[...] = jnp.maximum(ms_ref[...], bs)
        md_ref[...] = jnp.maximum(md_ref[...], bd)

    @pl.when(i == nb - 1)
    def _():
        t = ms_ref[...] + md_ref[...]
        m_ref[...] = jnp.where(t > 0.0, t, 0.2 * t)


def _prologue(x, W_gat, att_src, att_dst):
    n, in_dim = x.shape
    hh, cc = att_src.shape
    blk = 1000
    grid = n // blk
    return pl.pallas_call(
        functools.partial(_prologue_body, hh, cc),
        grid=(grid,),
        in_specs=[
            pl.BlockSpec((blk, in_dim), lambda i: (i, 0)),
            pl.BlockSpec((in_dim, hh * cc), lambda i: (0, 0)),
            pl.BlockSpec((hh, cc), lambda i: (0, 0)),
            pl.BlockSpec((hh, cc), lambda i: (0, 0)),
        ],
        out_specs=[
            pl.BlockSpec((blk, hh), lambda i: (i, 0)),
            pl.BlockSpec((blk, hh), lambda i: (i, 0)),
            pl.BlockSpec((1, hh), lambda i: (0, 0)),
        ],
        out_shape=[
            jax.ShapeDtypeStruct((n, hh), jnp.float32),
            jax.ShapeDtypeStruct((n, hh), jnp.float32),
            jax.ShapeDtypeStruct((1, hh), jnp.float32),
        ],
        scratch_shapes=[
            pltpu.VMEM((1, hh), jnp.float32),
            pltpu.VMEM((1, hh), jnp.float32),
        ],
    )(x, W_gat, att_src, att_dst)


def _make_sc_edge(n, in_dim, hh, k_chunks, nrow, adst_len):
    mesh = plsc.VectorSubcoreMesh(core_axis_name="c", subcore_axis_name="s",
                                  num_cores=NC, num_subcores=NS)
    rows_per = nrow // NS
    npay = hh * in_dim + hh

    @functools.partial(
        pl.kernel,
        out_type=jax.ShapeDtypeStruct((NC, nrow, ZW), jnp.float32),
        mesh=mesh,
        compiler_params=pltpu.CompilerParams(use_tc_tiling_on_sc=False,
                                             needs_layout_passes=False),
        scratch_types=[
            pltpu.VMEM((k_chunks, CH), jnp.int32),
            pltpu.VMEM((k_chunks, CH), jnp.int32),
            pltpu.VMEM((adst_len,), jnp.float32),
            pltpu.VMEM((CH, 16), jnp.float32),
            pltpu.VMEM((CH, ZW), jnp.float32),
            pltpu.VMEM((hh * 16,), jnp.float32),
            pltpu.VMEM_SHARED((nrow, ZW), jnp.float32),
            pltpu.SemaphoreType.DMA,
        ],
    )
    def sc_edge(src_hbm, dst_hbm, adst_hbm, tsrc_hbm, msp_hbm, zero_hbm,
                z_hbm, src_v, dst_v, adst_v, tsrc_v, pay_v, m_v, z_sh, sem):
        c = lax.axis_index("c")
        s = lax.axis_index("s")
        wid = s * NC + c
        pltpu.sync_copy(src_hbm.at[wid], src_v)
        pltpu.sync_copy(dst_hbm.at[wid], dst_v)
        pltpu.sync_copy(adst_hbm, adst_v)
        pltpu.sync_copy(msp_hbm, m_v)
        pltpu.sync_copy(zero_hbm.at[pl.ds(s * rows_per, rows_per)],
                        z_sh.at[pl.ds(s * rows_per, rows_per)])
        plsc.subcore_barrier()

        m_vecs = [m_v[pl.ds(h * 16, 16)] for h in range(hh)]
        lane = lax.iota(jnp.int32, 16)

        def chunk(j, carry):
            pltpu.async_copy(tsrc_hbm.at[src_v.at[j]], tsrc_v, sem).wait()
            for g in range(CH // 16):
                row16 = lane + (g * 16)
                dst16 = dst_v[j, pl.ds(g * 16, 16)]
                a_s = [plsc.load_gather(
                    tsrc_v, [row16, jnp.full((16,), h, jnp.int32)])
                    for h in range(hh)]
                xs = [plsc.load_gather(
                    tsrc_v, [row16, jnp.full((16,), hh + i, jnp.int32)])
                    for i in range(in_dim)]
                base = dst16 * hh
                ps = []
                for h in range(hh):
                    a_d = plsc.load_gather(adst_v, [base + h])
                    al = a_s[h] + a_d
                    al = jnp.where(al > 0.0, al, 0.2 * al)
                    p = jnp.exp(al - m_vecs[h])
                    ps.append(p)
                    plsc.store_scatter(
                        pay_v,
                        [row16, jnp.full((16,), hh * in_dim + h, jnp.int32)],
                        p)
                for h in range(hh):
                    for i in range(in_dim):
                        plsc.store_scatter(
                            pay_v,
                            [row16, jnp.full((16,), h * in_dim + i, jnp.int32)],
                            ps[h] * xs[i])
            pltpu.sync_copy(pay_v, z_sh.at[dst_v.at[j]], add=True)
            return carry

        lax.fori_loop(0, k_chunks, chunk, 0)
        plsc.subcore_barrier()
        pltpu.sync_copy(z_sh.at[pl.ds(s * rows_per, rows_per)],
                        z_hbm.at[c, pl.ds(s * rows_per, rows_per)])

    return sc_edge


def _epilogue_body(hh, cc, in_dim, act, z0_ref, z1_ref, xt_ref, ast_ref,
                   adt_ref, m_ref, wg_ref, bg_ref, w1_ref, b1_ref, w2_ref,
                   b2_ref, w3_ref, b3_ref, mean_ref, std_ref):
    zz = z0_ref[...] + z1_ref[...]
    al = ast_ref[...] + adt_ref[...]
    al = jnp.where(al > 0.0, al, 0.2 * al)
    p = jnp.exp(al - m_ref[...])
    xv = xt_ref[...]
    wg = wg_ref[...]
    embs = []
    for h in range(hh):
        zh = zz[:, h * in_dim:(h + 1) * in_dim] + p[:, h:h + 1] * xv
        den = zz[:, hh * in_dim + h:hh * in_dim + h + 1] + p[:, h:h + 1]
        zh = zh / (den + 1e-16)
        embs.append(jnp.dot(zh, wg[:, h * cc:(h + 1) * cc],
                            preferred_element_type=jnp.float32))
    emb = jnp.concatenate(embs, axis=1) + bg_ref[...]
    hdn = jnp.maximum(
        jnp.dot(emb, w1_ref[...], preferred_element_type=jnp.float32)
        + b1_ref[...], 0.0)
    hdn = jnp.maximum(
        jnp.dot(hdn, w2_ref[...], preferred_element_type=jnp.float32)
        + b2_ref[...], 0.0)
    a_out = jnp.dot(hdn, w3_ref[...], preferred_element_type=jnp.float32) \
        + b3_ref[...]
    mean_ref[...] = a_out[:, :act]
    log_std = jnp.clip(a_out[:, act:], -20.0, 2.0)
    std_ref[...] = jnp.exp(log_std)


def _epilogue(z0, z1, xt, ast, adt, m, W_gat, b_gat, aW1, ab1, aW2, ab2,
              aW3, ab3):
    nt, in_dim = xt.shape
    hh = ast.shape[1]
    cc = W_gat.shape[1] // hh
    hid = aW1.shape[1]
    act = aW3.shape[1] // 2
    blk = 1000
    grid = nt // blk
    c0 = lambda i: (0, 0)
    return pl.pallas_call(
        functools.partial(_epilogue_body, hh, cc, in_dim, act),
        grid=(grid,),
        in_specs=[
            pl.BlockSpec((blk, ZW), lambda i: (i, 0)),
            pl.BlockSpec((blk, ZW), lambda i: (i, 0)),
            pl.BlockSpec((blk, in_dim), lambda i: (i, 0)),
            pl.BlockSpec((blk, hh), lambda i: (i, 0)),
            pl.BlockSpec((blk, hh), lambda i: (i, 0)),
            pl.BlockSpec((1, hh), c0),
            pl.BlockSpec((in_dim, hh * cc), c0),
            pl.BlockSpec((1, hh * cc), c0),
            pl.BlockSpec((hh * cc, hid), c0),
            pl.BlockSpec((1, hid), c0),
            pl.BlockSpec((hid, hid), c0),
            pl.BlockSpec((1, hid), c0),
            pl.BlockSpec((hid, 2 * act), c0),
            pl.BlockSpec((1, 2 * act), c0),
        ],
        out_specs=[
            pl.BlockSpec((blk, act), lambda i: (i, 0)),
            pl.BlockSpec((blk, act), lambda i: (i, 0)),
        ],
        out_shape=[
            jax.ShapeDtypeStruct((nt, act), jnp.float32),
            jax.ShapeDtypeStruct((nt, act), jnp.float32),
        ],
    )(z0, z1, xt, ast, adt, m, W_gat, b_gat.reshape(1, -1), aW1,
      ab1.reshape(1, -1), aW2, ab2.reshape(1, -1), aW3, ab3.reshape(1, -1))


def _critic_body(obs_ref, w1_ref, b1_ref, w2_ref, b2_ref, w3_ref, b3_ref,
                 out_ref):
    v = jnp.maximum(
        jnp.dot(obs_ref[...], w1_ref[...], preferred_element_type=jnp.float32)
        + b1_ref[...], 0.0)
    v = jnp.maximum(
        jnp.dot(v, w2_ref[...], preferred_element_type=jnp.float32)
        + b2_ref[...], 0.0)
    out_ref[...] = jnp.dot(v, w3_ref[...],
                           preferred_element_type=jnp.float32) + b3_ref[...]


def _critic(obs, cW1, cb1, cW2, cb2, cW3, cb3):
    b, in_dim = obs.shape
    hid = cW1.shape[1]
    return pl.pallas_call(
        _critic_body,
        out_shape=jax.ShapeDtypeStruct((b, 1), jnp.float32),
    )(obs, cW1, cb1.reshape(1, -1), cW2, cb2.reshape(1, -1), cW3,
      cb3.reshape(1, -1))


def kernel(x, edge_index, observation, W_gat, att_src, att_dst, b_gat,
           aW1, ab1, aW2, ab2, aW3, ab3, cW1, cb1, cW2, cb2, cW3, cb3):
    n, in_dim = x.shape
    hh, cc = att_src.shape
    e = edge_index.shape[1]

    a_src, a_dst, m = _prologue(x, W_gat, att_src, att_dst)

    k_chunks = -(-e // (NW * CH))
    e_pad = NW * CH * k_chunks
    idt = edge_index.dtype
    src = jnp.concatenate(
        [edge_index[0], jnp.zeros((e_pad - e,), idt)]).reshape(NW, k_chunks, CH)
    dst = jnp.concatenate(
        [edge_index[1], jnp.full((e_pad - e,), n, idt)]).reshape(NW, k_chunks, CH)
    src = src.astype(jnp.int32)
    dst = dst.astype(jnp.int32)

    nrow = (n + NS + 127) // 128 * 128
    adst_len = -(-((n + 1) * hh) // 64) * 64
    adst_flat = jnp.zeros((adst_len,), jnp.float32)
    adst_flat = adst_flat.at[:n * hh].set(a_dst.reshape(-1))
    tsrc = jnp.concatenate(
        [a_src, x, jnp.zeros((n, 16 - hh - in_dim), jnp.float32)], axis=1)
    msp = jnp.broadcast_to(m.reshape(hh, 1), (hh, 16)).reshape(-1)
    zeros = jnp.zeros((nrow, ZW), jnp.float32)

    sc_edge = _make_sc_edge(n, in_dim, hh, k_chunks, nrow, adst_len)
    z = sc_edge(src, dst, adst_flat, tsrc, msp, zeros)

    ng = n // TEAM_MOD
    zt = z[:, :n, :].reshape(NC, ng, TEAM_MOD, ZW)[:, :, :TEAM_KEEP, :]
    zt = zt.reshape(NC, ng * TEAM_KEEP, ZW)
    sel = lambda a: a.reshape(ng, TEAM_MOD, -1)[:, :TEAM_KEEP].reshape(
        ng * TEAM_KEEP, -1)
    xt, ast, adt = sel(x), sel(a_src), sel(a_dst)

    mean, std = _epilogue(zt[0], zt[1], xt, ast, adt, m, W_gat, b_gat,
                          aW1, ab1, aW2, ab2, aW3, ab3)
    value = _critic(observation, cW1, cb1, cW2, cb2, cW3, cb3)
    return (mean, std, value)


# speedup vs baseline: 117.7148x; 1.1966x over previous
import functools

import jax
import jax.numpy as jnp
from jax import lax
from jax.experimental import pallas as pl
from jax.experimental.pallas import tpu as pltpu
from jax.experimental.pallas import tpu_sc as plsc

NC = 2
NS = 16
NW = NC * NS
CH = 128
ZW = 32
TEAM_MOD = 10
TEAM_KEEP = 5


def _prologue_body(H, C, x_ref, w_ref, ats_ref, atd_ref,
                   asrc_ref, adst_ref, m_ref, ms_ref, md_ref):
    i = pl.program_id(0)
    nb = pl.num_programs(0)
    blk = x_ref.shape[0]
    h = jnp.dot(x_ref[...], w_ref[...], preferred_element_type=jnp.float32)
    h3 = h.reshape(blk, H, C)
    a_s = jnp.sum(h3 * ats_ref[...][None], axis=-1)
    a_d = jnp.sum(h3 * atd_ref[...][None], axis=-1)
    asrc_ref[...] = a_s
    adst_ref[...] = a_d
    bs = jnp.max(a_s, axis=0, keepdims=True)
    bd = jnp.max(a_d, axis=0, keepdims=True)

    @pl.when(i == 0)
    def _():
        ms_ref[...] = bs
        md_ref[...] = bd

    @pl.when(i > 0)
    def _():
        ms_ref[...] = jnp.maximum(ms_ref[...], bs)
        md_ref[...] = jnp.maximum(md_ref[...], bd)

    @pl.when(i == nb - 1)
    def _():
        t = ms_ref[...] + md_ref[...]
        m_ref[...] = jnp.where(t > 0.0, t, 0.2 * t)


def _prologue(x, W_gat, att_src, att_dst):
    n, in_dim = x.shape
    hh, cc = att_src.shape
    blk = 1000
    grid = n // blk
    return pl.pallas_call(
        functools.partial(_prologue_body, hh, cc),
        grid=(grid,),
        in_specs=[
            pl.BlockSpec((blk, in_dim), lambda i: (i, 0)),
            pl.BlockSpec((in_dim, hh * cc), lambda i: (0, 0)),
            pl.BlockSpec((hh, cc), lambda i: (0, 0)),
            pl.BlockSpec((hh, cc), lambda i: (0, 0)),
        ],
        out_specs=[
            pl.BlockSpec((blk, hh), lambda i: (i, 0)),
            pl.BlockSpec((blk, hh), lambda i: (i, 0)),
            pl.BlockSpec((1, hh), lambda i: (0, 0)),
        ],
        out_shape=[
            jax.ShapeDtypeStruct((n, hh), jnp.float32),
            jax.ShapeDtypeStruct((n, hh), jnp.float32),
            jax.ShapeDtypeStruct((1, hh), jnp.float32),
        ],
        scratch_shapes=[
            pltpu.VMEM((1, hh), jnp.float32),
            pltpu.VMEM((1, hh), jnp.float32),
        ],
    )(x, W_gat, att_src, att_dst)


def _make_sc_edge(n, in_dim, hh, k_chunks, nrow, adst_len):
    mesh = plsc.VectorSubcoreMesh(core_axis_name="c", subcore_axis_name="s",
                                  num_cores=NC, num_subcores=NS)
    rows_per = nrow // NS
    npay = hh * in_dim + hh

    @functools.partial(
        pl.kernel,
        out_type=jax.ShapeDtypeStruct((NC, nrow, ZW), jnp.float32),
        mesh=mesh,
        compiler_params=pltpu.CompilerParams(use_tc_tiling_on_sc=False,
                                             needs_layout_passes=False),
        scratch_types=[
            pltpu.VMEM((k_chunks, CH), jnp.int32),
            pltpu.VMEM((k_chunks, CH), jnp.int32),
            pltpu.VMEM((adst_len,), jnp.float32),
            pltpu.VMEM((CH, 16), jnp.float32),
            pltpu.VMEM((CH, 16), jnp.float32),
            pltpu.VMEM((CH, ZW), jnp.float32),
            pltpu.VMEM((CH, ZW), jnp.float32),
            pltpu.VMEM((hh * 16,), jnp.float32),
            pltpu.VMEM_SHARED((nrow, ZW), jnp.float32),
            pltpu.SemaphoreType.DMA,
            pltpu.SemaphoreType.DMA,
            pltpu.SemaphoreType.DMA,
            pltpu.SemaphoreType.DMA,
        ],
    )
    def sc_edge(src_hbm, dst_hbm, adst_hbm, tsrc_hbm, msp_hbm, zero_hbm,
                z_hbm, src_v, dst_v, adst_v, tsrcA, tsrcB, payA, payB,
                m_v, z_sh, gsA, gsB, ssA, ssB):
        c = lax.axis_index("c")
        s = lax.axis_index("s")
        wid = s * NC + c
        pltpu.sync_copy(src_hbm.at[wid], src_v)
        pltpu.sync_copy(dst_hbm.at[wid], dst_v)
        pltpu.sync_copy(adst_hbm, adst_v)
        pltpu.sync_copy(msp_hbm, m_v)
        pltpu.sync_copy(zero_hbm.at[pl.ds(s * rows_per, rows_per)],
                        z_sh.at[pl.ds(s * rows_per, rows_per)])
        plsc.subcore_barrier()

        m_vecs = [m_v[pl.ds(h * 16, 16)] for h in range(hh)]
        lane = lax.iota(jnp.int32, 16)
        cols = [jnp.full((16,), col, jnp.int32) for col in range(ZW)]

        def compute(j, tsrc_v, pay_v):
            for g in range(CH // 16):
                row16 = lane + (g * 16)
                dst16 = dst_v[j, pl.ds(g * 16, 16)]
                a_s = [plsc.load_gather(tsrc_v, [row16, cols[h]])
                       for h in range(hh)]
                xs = [plsc.load_gather(tsrc_v, [row16, cols[hh + i]])
                      for i in range(in_dim)]
                base = dst16 * hh
                ps = []
                for h in range(hh):
                    a_d = plsc.load_gather(adst_v, [base + h])
                    al = a_s[h] + a_d
                    al = jnp.where(al > 0.0, al, 0.2 * al)
                    p = jnp.exp(al - m_vecs[h])
                    ps.append(p)
                    plsc.store_scatter(pay_v, [row16, cols[hh * in_dim + h]],
                                       p)
                for h in range(hh):
                    for i in range(in_dim):
                        plsc.store_scatter(pay_v,
                                           [row16, cols[h * in_dim + i]],
                                           ps[h] * xs[i])

        pltpu.async_copy(tsrc_hbm.at[src_v.at[0]], tsrcA, gsA)
        pltpu.async_copy(tsrc_hbm.at[src_v.at[1]], tsrcB, gsB)

        def body(j2, carry):
            for par, (buf, pay, gs, ss) in enumerate(
                    [(tsrcA, payA, gsA, ssA), (tsrcB, payB, gsB, ssB)]):
                j = j2 * 2 + par

                @pl.when(j2 > 0)
                def _():
                    pltpu.make_async_copy(z_hbm.at[0, pl.ds(0, CH)], pay,
                                          ss).wait()

                pltpu.make_async_copy(tsrc_hbm.at[pl.ds(0, CH)], buf,
                                      gs).wait()
                compute(j, buf, pay)

                @pl.when(j + 2 < k_chunks)
                def _():
                    pltpu.async_copy(tsrc_hbm.at[src_v.at[j + 2]], buf, gs)

                pltpu.async_copy(pay, z_sh.at[dst_v.at[j]], ss, add=True)
            return carry

        lax.fori_loop(0, k_chunks // 2, body, 0)
        pltpu.make_async_copy(z_hbm.at[0, pl.ds(0, CH)], payA, ssA).wait()
        pltpu.make_async_copy(z_hbm.at[0, pl.ds(0, CH)], payB, ssB).wait()
        plsc.subcore_barrier()
        pltpu.sync_copy(z_sh.at[pl.ds(s * rows_per, rows_per)],
                        z_hbm.at[c, pl.ds(s * rows_per, rows_per)])

    return sc_edge


def _epilogue_body(hh, cc, in_dim, act, z0_ref, z1_ref, xt_ref, ast_ref,
                   adt_ref, m_ref, wg_ref, bg_ref, w1_ref, b1_ref, w2_ref,
                   b2_ref, w3_ref, b3_ref, mean_ref, std_ref):
    zz = z0_ref[...] + z1_ref[...]
    al = ast_ref[...] + adt_ref[...]
    al = jnp.where(al > 0.0, al, 0.2 * al)
    p = jnp.exp(al - m_ref[...])
    xv = xt_ref[...]
    wg = wg_ref[...]
    embs = []
    for h in range(hh):
        zh = zz[:, h * in_dim:(h + 1) * in_dim] + p[:, h:h + 1] * xv
        den = zz[:, hh * in_dim + h:hh * in_dim + h + 1] + p[:, h:h + 1]
        zh = zh / (den + 1e-16)
        embs.append(jnp.dot(zh, wg[:, h * cc:(h + 1) * cc],
                            preferred_element_type=jnp.float32))
    emb = jnp.concatenate(embs, axis=1) + bg_ref[...]
    hdn = jnp.maximum(
        jnp.dot(emb, w1_ref[...], preferred_element_type=jnp.float32)
        + b1_ref[...], 0.0)
    hdn = jnp.maximum(
        jnp.dot(hdn, w2_ref[...], preferred_element_type=jnp.float32)
        + b2_ref[...], 0.0)
    a_out = jnp.dot(hdn, w3_ref[...], preferred_element_type=jnp.float32) \
        + b3_ref[...]
    mean_ref[...] = a_out[:, :act]
    log_std = jnp.clip(a_out[:, act:], -20.0, 2.0)
    std_ref[...] = jnp.exp(log_std)


def _epilogue(z0, z1, xt, ast, adt, m, W_gat, b_gat, aW1, ab1, aW2, ab2,
              aW3, ab3):
    nt, in_dim = xt.shape
    hh = ast.shape[1]
    cc = W_gat.shape[1] // hh
    hid = aW1.shape[1]
    act = aW3.shape[1] // 2
    blk = 1000
    grid = nt // blk
    c0 = lambda i: (0, 0)
    return pl.pallas_call(
        functools.partial(_epilogue_body, hh, cc, in_dim, act),
        grid=(grid,),
        in_specs=[
            pl.BlockSpec((blk, ZW), lambda i: (i, 0)),
            pl.BlockSpec((blk, ZW), lambda i: (i, 0)),
            pl.BlockSpec((blk, in_dim), lambda i: (i, 0)),
            pl.BlockSpec((blk, hh), lambda i: (i, 0)),
            pl.BlockSpec((blk, hh), lambda i: (i, 0)),
            pl.BlockSpec((1, hh), c0),
            pl.BlockSpec((in_dim, hh * cc), c0),
            pl.BlockSpec((1, hh * cc), c0),
            pl.BlockSpec((hh * cc, hid), c0),
            pl.BlockSpec((1, hid), c0),
            pl.BlockSpec((hid, hid), c0),
            pl.BlockSpec((1, hid), c0),
            pl.BlockSpec((hid, 2 * act), c0),
            pl.BlockSpec((1, 2 * act), c0),
        ],
        out_specs=[
            pl.BlockSpec((blk, act), lambda i: (i, 0)),
            pl.BlockSpec((blk, act), lambda i: (i, 0)),
        ],
        out_shape=[
            jax.ShapeDtypeStruct((nt, act), jnp.float32),
            jax.ShapeDtypeStruct((nt, act), jnp.float32),
        ],
    )(z0, z1, xt, ast, adt, m, W_gat, b_gat.reshape(1, -1), aW1,
      ab1.reshape(1, -1), aW2, ab2.reshape(1, -1), aW3, ab3.reshape(1, -1))


def _critic_body(obs_ref, w1_ref, b1_ref, w2_ref, b2_ref, w3_ref, b3_ref,
                 out_ref):
    v = jnp.maximum(
        jnp.dot(obs_ref[...], w1_ref[...], preferred_element_type=jnp.float32)
        + b1_ref[...], 0.0)
    v = jnp.maximum(
        jnp.dot(v, w2_ref[...], preferred_element_type=jnp.float32)
        + b2_ref[...], 0.0)
    out_ref[...] = jnp.dot(v, w3_ref[...],
                           preferred_element_type=jnp.float32) + b3_ref[...]


def _critic(obs, cW1, cb1, cW2, cb2, cW3, cb3):
    b, in_dim = obs.shape
    hid = cW1.shape[1]
    return pl.pallas_call(
        _critic_body,
        out_shape=jax.ShapeDtypeStruct((b, 1), jnp.float32),
    )(obs, cW1, cb1.reshape(1, -1), cW2, cb2.reshape(1, -1), cW3,
      cb3.reshape(1, -1))


def kernel(x, edge_index, observation, W_gat, att_src, att_dst, b_gat,
           aW1, ab1, aW2, ab2, aW3, ab3, cW1, cb1, cW2, cb2, cW3, cb3):
    n, in_dim = x.shape
    hh, cc = att_src.shape
    e = edge_index.shape[1]

    a_src, a_dst, m = _prologue(x, W_gat, att_src, att_dst)

    k_chunks = -(-e // (NW * CH))
    k_chunks += k_chunks % 2
    e_pad = NW * CH * k_chunks
    idt = edge_index.dtype
    src = jnp.concatenate(
        [edge_index[0], jnp.zeros((e_pad - e,), idt)]).reshape(NW, k_chunks, CH)
    dst = jnp.concatenate(
        [edge_index[1], jnp.full((e_pad - e,), n, idt)]).reshape(NW, k_chunks, CH)
    src = src.astype(jnp.int32)
    dst = dst.astype(jnp.int32)

    nrow = (n + NS + 127) // 128 * 128
    adst_len = -(-((n + 1) * hh) // 64) * 64
    adst_flat = jnp.zeros((adst_len,), jnp.float32)
    adst_flat = adst_flat.at[:n * hh].set(a_dst.reshape(-1))
    tsrc = jnp.concatenate(
        [a_src, x, jnp.zeros((n, 16 - hh - in_dim), jnp.float32)], axis=1)
    msp = jnp.broadcast_to(m.reshape(hh, 1), (hh, 16)).reshape(-1)
    zeros = jnp.zeros((nrow, ZW), jnp.float32)

    sc_edge = _make_sc_edge(n, in_dim, hh, k_chunks, nrow, adst_len)
    z = sc_edge(src, dst, adst_flat, tsrc, msp, zeros)

    ng = n // TEAM_MOD
    zt = z[:, :n, :].reshape(NC, ng, TEAM_MOD, ZW)[:, :, :TEAM_KEEP, :]
    zt = zt.reshape(NC, ng * TEAM_KEEP, ZW)
    sel = lambda a: a.reshape(ng, TEAM_MOD, -1)[:, :TEAM_KEEP].reshape(
        ng * TEAM_KEEP, -1)
    xt, ast, adt = sel(x), sel(a_src), sel(a_dst)

    mean, std = _epilogue(zt[0], zt[1], xt, ast, adt, m, W_gat, b_gat,
                          aW1, ab1, aW2, ab2, aW3, ab3)
    value = _critic(observation, cW1, cb1, cW2, cb2, cW3, cb3)
    return (mean, std, value)


# speedup vs baseline: 126.1071x; 1.0713x over previous
import functools

import jax
import jax.numpy as jnp
from jax import lax
from jax.experimental import pallas as pl
from jax.experimental.pallas import tpu as pltpu
from jax.experimental.pallas import tpu_sc as plsc

NC = 2
NS = 16
NW = NC * NS
CH = 128
ZW = 32
TEAM_MOD = 10
TEAM_KEEP = 5


def _prologue_body(H, C, x_ref, w_ref, ats_ref, atd_ref,
                   asrc_ref, adst_ref, m_ref, ms_ref, md_ref):
    i = pl.program_id(0)
    nb = pl.num_programs(0)
    blk = x_ref.shape[0]
    h = jnp.dot(x_ref[...], w_ref[...], preferred_element_type=jnp.float32)
    h3 = h.reshape(blk, H, C)
    a_s = jnp.sum(h3 * ats_ref[...][None], axis=-1)
    a_d = jnp.sum(h3 * atd_ref[...][None], axis=-1)
    asrc_ref[...] = a_s
    adst_ref[...] = a_d
    bs = jnp.max(a_s, axis=0, keepdims=True)
    bd = jnp.max(a_d, axis=0, keepdims=True)

    @pl.when(i == 0)
    def _():
        ms_ref[...] = bs
        md_ref[...] = bd

    @pl.when(i > 0)
    def _():
        ms_ref[...] = jnp.maximum(ms_ref[...], bs)
        md_ref[...] = jnp.maximum(md_ref[...], bd)

    @pl.when(i == nb - 1)
    def _():
        t = ms_ref[...] + md_ref[...]
        m_ref[...] = jnp.where(t > 0.0, t, 0.2 * t)


def _prologue(x, W_gat, att_src, att_dst):
    n, in_dim = x.shape
    hh, cc = att_src.shape
    blk = 1000
    grid = n // blk
    return pl.pallas_call(
        functools.partial(_prologue_body, hh, cc),
        grid=(grid,),
        in_specs=[
            pl.BlockSpec((blk, in_dim), lambda i: (i, 0)),
            pl.BlockSpec((in_dim, hh * cc), lambda i: (0, 0)),
            pl.BlockSpec((hh, cc), lambda i: (0, 0)),
            pl.BlockSpec((hh, cc), lambda i: (0, 0)),
        ],
        out_specs=[
            pl.BlockSpec((blk, hh), lambda i: (i, 0)),
            pl.BlockSpec((blk, hh), lambda i: (i, 0)),
            pl.BlockSpec((1, hh), lambda i: (0, 0)),
        ],
        out_shape=[
            jax.ShapeDtypeStruct((n, hh), jnp.float32),
            jax.ShapeDtypeStruct((n, hh), jnp.float32),
            jax.ShapeDtypeStruct((1, hh), jnp.float32),
        ],
        scratch_shapes=[
            pltpu.VMEM((1, hh), jnp.float32),
            pltpu.VMEM((1, hh), jnp.float32),
        ],
    )(x, W_gat, att_src, att_dst)


def _make_sc_edge(n, in_dim, hh, k_chunks, nrow, np1, sh):
    mesh = plsc.VectorSubcoreMesh(core_axis_name="c", subcore_axis_name="s",
                                  num_cores=NC, num_subcores=NS)
    rows_per = nrow // NS
    smask = (1 << sh) - 1

    @functools.partial(
        pl.kernel,
        out_type=jax.ShapeDtypeStruct((NC, nrow, ZW), jnp.float32),
        mesh=mesh,
        compiler_params=pltpu.CompilerParams(use_tc_tiling_on_sc=False,
                                             needs_layout_passes=False),
        scratch_types=[
            pltpu.VMEM((hh * np1,), jnp.float32),
            pltpu.VMEM(((hh // 2) * np1,), jnp.int32),
            pltpu.VMEM((3 * np1,), jnp.int32),
            pltpu.VMEM((1, CH), jnp.int32),
            pltpu.VMEM((1, CH), jnp.int32),
            pltpu.VMEM((1, CH), jnp.int32),
            pltpu.VMEM((1, CH), jnp.int32),
            pltpu.VMEM((CH, ZW), jnp.float32),
            pltpu.VMEM((CH, ZW), jnp.float32),
            pltpu.VMEM((hh * 16,), jnp.float32),
            pltpu.VMEM_SHARED((nrow, ZW), jnp.float32),
            pltpu.SemaphoreType.DMA,
            pltpu.SemaphoreType.DMA,
            pltpu.SemaphoreType.DMA,
            pltpu.SemaphoreType.DMA,
        ],
    )
    def sc_edge(edges_hbm, asrc_hbm, adst_hbm, xp_hbm, msp_hbm, zero_hbm,
                z_hbm, asrc_v, adst_v, xp_v, ibufA, ibufB, didxA, didxB,
                payA, payB, m_v, z_sh, gsA, gsB, ssA, ssB):
        c = lax.axis_index("c")
        s = lax.axis_index("s")
        wid = s * NC + c
        pltpu.sync_copy(asrc_hbm, asrc_v)
        pltpu.sync_copy(adst_hbm, adst_v)
        pltpu.sync_copy(xp_hbm, xp_v)
        pltpu.sync_copy(msp_hbm, m_v)
        pltpu.sync_copy(zero_hbm, z_sh.at[pl.ds(s * rows_per, rows_per)])
        plsc.subcore_barrier()

        m_vecs = [m_v[pl.ds(h * 16, 16)] for h in range(hh)]
        lane = lax.iota(jnp.int32, 16)
        cols = [jnp.full((16,), col, jnp.int32) for col in range(ZW)]
        offs = [jnp.full((16,), h * np1, jnp.int32) for h in range(hh)]

        def compute(ibuf, didx, pay_v):
            for g in range(CH // 16):
                row16 = lane + (g * 16)
                w16 = ibuf[0, pl.ds(g * 16, 16)]
                src16 = w16 & smask
                dst16 = jax.lax.shift_right_logical(w16, sh)
                didx[0, pl.ds(g * 16, 16)] = dst16
                a_s = [plsc.load_gather(asrc_v, [src16 + offs[h]])
                       for h in range(hh)]
                ad01r = plsc.load_gather(adst_v, [dst16])
                ad23r = plsc.load_gather(adst_v, [dst16 + offs[1]])
                ad0, ad1 = plsc.unpack(plsc.bitcast(ad01r, jnp.bfloat16),
                                       format=plsc.PackFormat.INTERLEAVED,
                                       preferred_element_type=jnp.float32)
                ad2, ad3 = plsc.unpack(plsc.bitcast(ad23r, jnp.bfloat16),
                                       format=plsc.PackFormat.INTERLEAVED,
                                       preferred_element_type=jnp.float32)
                a_d = [ad0, ad1, ad2, ad3]
                x01r = plsc.load_gather(xp_v, [src16])
                x23r = plsc.load_gather(xp_v, [src16 + offs[1]])
                x4r = plsc.load_gather(xp_v, [src16 + offs[2]])
                x0, x1 = plsc.unpack(plsc.bitcast(x01r, jnp.bfloat16),
                                     format=plsc.PackFormat.INTERLEAVED,
                                     preferred_element_type=jnp.float32)
                x2, x3 = plsc.unpack(plsc.bitcast(x23r, jnp.bfloat16),
                                     format=plsc.PackFormat.INTERLEAVED,
                                     preferred_element_type=jnp.float32)
                xs = [x0, x1, x2, x3, plsc.bitcast(x4r, jnp.float32)]
                for h in range(hh):
                    al = a_s[h] + a_d[h]
                    al = jnp.where(al > 0.0, al, 0.2 * al)
                    p = jnp.exp(al - m_vecs[h])
                    plsc.store_scatter(pay_v, [row16, cols[hh * in_dim + h]],
                                       p)
                    for i in range(in_dim):
                        plsc.store_scatter(pay_v,
                                           [row16, cols[h * in_dim + i]],
                                           p * xs[i])

        pltpu.async_copy(edges_hbm.at[wid, 0], ibufA.at[0], gsA)
        pltpu.async_copy(edges_hbm.at[wid, 1], ibufB.at[0], gsB)

        def body(j2, carry):
            for par, (ibuf, didx, pay, gs, ss) in enumerate(
                    [(ibufA, didxA, payA, gsA, ssA),
                     (ibufB, didxB, payB, gsB, ssB)]):
                j = j2 * 2 + par

                @pl.when(j2 > 0)
                def _():
                    pltpu.make_async_copy(z_hbm.at[0, pl.ds(0, CH)], pay,
                                          ss).wait()

                pltpu.make_async_copy(edges_hbm.at[0, 0], ibuf.at[0],
                                      gs).wait()
                compute(ibuf, didx, pay)

                @pl.when(j + 2 < k_chunks)
                def _():
                    pltpu.async_copy(edges_hbm.at[wid, j + 2], ibuf.at[0], gs)

                pltpu.async_copy(pay, z_sh.at[didx.at[0]], ss, add=True)
            return carry

        lax.fori_loop(0, k_chunks // 2, body, 0)
        pltpu.make_async_copy(z_hbm.at[0, pl.ds(0, CH)], payA, ssA).wait()
        pltpu.make_async_copy(z_hbm.at[0, pl.ds(0, CH)], payB, ssB).wait()
        plsc.subcore_barrier()
        pltpu.sync_copy(z_sh.at[pl.ds(s * rows_per, rows_per)],
                        z_hbm.at[c, pl.ds(s * rows_per, rows_per)])

    return sc_edge


def _epilogue_body(hh, cc, in_dim, act, z0_ref, z1_ref, xt_ref, ast_ref,
                   adt_ref, m_ref, wg_ref, bg_ref, w1_ref, b1_ref, w2_ref,
                   b2_ref, w3_ref, b3_ref, mean_ref, std_ref):
    zz = z0_ref[...] + z1_ref[...]
    al = ast_ref[...] + adt_ref[...]
    al = jnp.where(al > 0.0, al, 0.2 * al)
    p = jnp.exp(al - m_ref[...])
    xv = xt_ref[...]
    wg = wg_ref[...]
    embs = []
    for h in range(hh):
        zh = zz[:, h * in_dim:(h + 1) * in_dim] + p[:, h:h + 1] * xv
        den = zz[:, hh * in_dim + h:hh * in_dim + h + 1] + p[:, h:h + 1]
        zh = zh / (den + 1e-16)
        embs.append(jnp.dot(zh, wg[:, h * cc:(h + 1) * cc],
                            preferred_element_type=jnp.float32))
    emb = jnp.concatenate(embs, axis=1) + bg_ref[...]
    hdn = jnp.maximum(
        jnp.dot(emb, w1_ref[...], preferred_element_type=jnp.float32)
        + b1_ref[...], 0.0)
    hdn = jnp.maximum(
        jnp.dot(hdn, w2_ref[...], preferred_element_type=jnp.float32)
        + b2_ref[...], 0.0)
    a_out = jnp.dot(hdn, w3_ref[...], preferred_element_type=jnp.float32) \
        + b3_ref[...]
    mean_ref[...] = a_out[:, :act]
    log_std = jnp.clip(a_out[:, act:], -20.0, 2.0)
    std_ref[...] = jnp.exp(log_std)


def _epilogue(z0, z1, xt, ast, adt, m, W_gat, b_gat, aW1, ab1, aW2, ab2,
              aW3, ab3):
    nt, in_dim = xt.shape
    hh = ast.shape[1]
    cc = W_gat.shape[1] // hh
    hid = aW1.shape[1]
    act = aW3.shape[1] // 2
    blk = 1000
    grid = nt // blk
    c0 = lambda i: (0, 0)
    return pl.pallas_call(
        functools.partial(_epilogue_body, hh, cc, in_dim, act),
        grid=(grid,),
        in_specs=[
            pl.BlockSpec((blk, ZW), lambda i: (i, 0)),
            pl.BlockSpec((blk, ZW), lambda i: (i, 0)),
            pl.BlockSpec((blk, in_dim), lambda i: (i, 0)),
            pl.BlockSpec((blk, hh), lambda i: (i, 0)),
            pl.BlockSpec((blk, hh), lambda i: (i, 0)),
            pl.BlockSpec((1, hh), c0),
            pl.BlockSpec((in_dim, hh * cc), c0),
            pl.BlockSpec((1, hh * cc), c0),
            pl.BlockSpec((hh * cc, hid), c0),
            pl.BlockSpec((1, hid), c0),
            pl.BlockSpec((hid, hid), c0),
            pl.BlockSpec((1, hid), c0),
            pl.BlockSpec((hid, 2 * act), c0),
            pl.BlockSpec((1, 2 * act), c0),
        ],
        out_specs=[
            pl.BlockSpec((blk, act), lambda i: (i, 0)),
            pl.BlockSpec((blk, act), lambda i: (i, 0)),
        ],
        out_shape=[
            jax.ShapeDtypeStruct((nt, act), jnp.float32),
            jax.ShapeDtypeStruct((nt, act), jnp.float32),
        ],
    )(z0, z1, xt, ast, adt, m, W_gat, b_gat.reshape(1, -1), aW1,
      ab1.reshape(1, -1), aW2, ab2.reshape(1, -1), aW3, ab3.reshape(1, -1))


def _critic_body(obs_ref, w1_ref, b1_ref, w2_ref, b2_ref, w3_ref, b3_ref,
                 out_ref):
    v = jnp.maximum(
        jnp.dot(obs_ref[...], w1_ref[...], preferred_element_type=jnp.float32)
        + b1_ref[...], 0.0)
    v = jnp.maximum(
        jnp.dot(v, w2_ref[...], preferred_element_type=jnp.float32)
        + b2_ref[...], 0.0)
    out_ref[...] = jnp.dot(v, w3_ref[...],
                           preferred_element_type=jnp.float32) + b3_ref[...]


def _critic(obs, cW1, cb1, cW2, cb2, cW3, cb3):
    b, in_dim = obs.shape
    hid = cW1.shape[1]
    return pl.pallas_call(
        _critic_body,
        out_shape=jax.ShapeDtypeStruct((b, 1), jnp.float32),
    )(obs, cW1, cb1.reshape(1, -1), cW2, cb2.reshape(1, -1), cW3,
      cb3.reshape(1, -1))


def kernel(x, edge_index, observation, W_gat, att_src, att_dst, b_gat,
           aW1, ab1, aW2, ab2, aW3, ab3, cW1, cb1, cW2, cb2, cW3, cb3):
    n, in_dim = x.shape
    hh, cc = att_src.shape
    e = edge_index.shape[1]

    a_src, a_dst, m = _prologue(x, W_gat, att_src, att_dst)

    k_chunks = -(-e // (NW * CH))
    k_chunks += k_chunks % 2
    e_pad = NW * CH * k_chunks
    idt = edge_index.dtype
    src = jnp.concatenate(
        [edge_index[0], jnp.zeros((e_pad - e,), idt)]).reshape(NW, k_chunks, CH)
    dst = jnp.concatenate(
        [edge_index[1], jnp.full((e_pad - e,), n, idt)]).reshape(NW, k_chunks, CH)
    sh = (n + 1).bit_length()
    edges = (src.astype(jnp.int32)
             | (dst.astype(jnp.int32) << sh))

    nrow = (n + NS + 127) // 128 * 128
    np1 = -(-(n + 1) // 8) * 8
    pad1 = lambda a: jnp.pad(a, (0, np1 - n))
    asrc_flat = jnp.concatenate([pad1(a_src[:, h]) for h in range(hh)])
    pair = lambda a, b: (
        jax.lax.bitcast_convert_type(a.astype(jnp.bfloat16),
                                     jnp.uint16).astype(jnp.uint32)
        | (jax.lax.bitcast_convert_type(b.astype(jnp.bfloat16),
                                        jnp.uint16).astype(jnp.uint32) << 16)
    ).astype(jnp.int32)
    adst_flat = jnp.concatenate([pad1(pair(a_dst[:, 0], a_dst[:, 1])),
                                 pad1(pair(a_dst[:, 2], a_dst[:, 3]))])
    x4 = jax.lax.bitcast_convert_type(x[:, 4], jnp.int32)
    xp = jnp.concatenate([pad1(pair(x[:, 0], x[:, 1])),
                          pad1(pair(x[:, 2], x[:, 3])), pad1(x4)])
    msp = jnp.broadcast_to(m.reshape(hh, 1), (hh, 16)).reshape(-1)
    zeros = jnp.zeros((nrow // NS, ZW), jnp.float32)

    sc_edge = _make_sc_edge(n, in_dim, hh, k_chunks, nrow, np1, sh)
    z = sc_edge(edges, asrc_flat, adst_flat, xp, msp, zeros)

    ng = n // TEAM_MOD
    zt = z[:, :n, :].reshape(NC, ng, TEAM_MOD, ZW)[:, :, :TEAM_KEEP, :]
    zt = zt.reshape(NC, ng * TEAM_KEEP, ZW)
    sel = lambda a: a.reshape(ng, TEAM_MOD, -1)[:, :TEAM_KEEP].reshape(
        ng * TEAM_KEEP, -1)
    xt, ast, adt = sel(x), sel(a_src), sel(a_dst)

    mean, std = _epilogue(zt[0], zt[1], xt, ast, adt, m, W_gat, b_gat,
                          aW1, ab1, aW2, ab2, aW3, ab3)
    value = _critic(observation, cW1, cb1, cW2, cb2, cW3, cb3)
    return (mean, std, value)


# speedup vs baseline: 139.2755x; 1.1044x over previous
import functools

import jax
import jax.numpy as jnp
from jax import lax
from jax.experimental import pallas as pl
from jax.experimental.pallas import tpu as pltpu
from jax.experimental.pallas import tpu_sc as plsc

NC = 2
NS = 16
NW = NC * NS
CH = 128
ZW = 32
TEAM_MOD = 10
TEAM_KEEP = 5


def _prologue_body(H, C, x_ref, w_ref, ats_ref, atd_ref,
                   asrc_ref, adst_ref, m_ref):
    wg = w_ref[...]
    ws = jnp.concatenate(
        [jnp.sum(wg[:, h * C:(h + 1) * C] * ats_ref[h:h + 1, :], axis=1,
                 keepdims=True) for h in range(H)], axis=1)
    wd = jnp.concatenate(
        [jnp.sum(wg[:, h * C:(h + 1) * C] * atd_ref[h:h + 1, :], axis=1,
                 keepdims=True) for h in range(H)], axis=1)
    a_s = jnp.dot(x_ref[...], ws, preferred_element_type=jnp.float32)
    a_d = jnp.dot(x_ref[...], wd, preferred_element_type=jnp.float32)
    asrc_ref[...] = a_s
    adst_ref[...] = a_d
    t = (jnp.max(a_s, axis=0, keepdims=True)
         + jnp.max(a_d, axis=0, keepdims=True))
    m_ref[...] = jnp.where(t > 0.0, t, 0.2 * t)


def _prologue(x, W_gat, att_src, att_dst):
    n, in_dim = x.shape
    hh, cc = att_src.shape
    return pl.pallas_call(
        functools.partial(_prologue_body, hh, cc),
        out_shape=[
            jax.ShapeDtypeStruct((n, hh), jnp.float32),
            jax.ShapeDtypeStruct((n, hh), jnp.float32),
            jax.ShapeDtypeStruct((1, hh), jnp.float32),
        ],
    )(x, W_gat, att_src, att_dst)


def _make_sc_edge(n, in_dim, hh, ept, nrow, np1):
    mesh = plsc.VectorSubcoreMesh(core_axis_name="c", subcore_axis_name="s",
                                  num_cores=NC, num_subcores=NS)
    rows_per = nrow // NS
    kfull = ept // CH
    tail = ept % CH
    assert kfull % 2 == 0 and tail % 16 == 0 and tail < CH

    @functools.partial(
        pl.kernel,
        out_type=jax.ShapeDtypeStruct((NC, nrow, ZW), jnp.float32),
        mesh=mesh,
        compiler_params=pltpu.CompilerParams(use_tc_tiling_on_sc=False,
                                             needs_layout_passes=False),
        scratch_types=[
            pltpu.VMEM((hh * np1,), jnp.float32),
            pltpu.VMEM(((hh // 2) * np1,), jnp.int32),
            pltpu.VMEM((3 * np1,), jnp.int32),
            pltpu.VMEM((2, CH), jnp.int32),
            pltpu.VMEM((2, CH), jnp.int32),
            pltpu.VMEM((1, CH), jnp.int32),
            pltpu.VMEM((1, CH), jnp.int32),
            pltpu.VMEM((CH, ZW), jnp.float32),
            pltpu.VMEM((CH, ZW), jnp.float32),
            pltpu.VMEM((hh * 16,), jnp.float32),
            pltpu.VMEM_SHARED((nrow, ZW), jnp.float32),
            pltpu.SemaphoreType.DMA,
            pltpu.SemaphoreType.DMA,
            pltpu.SemaphoreType.DMA,
            pltpu.SemaphoreType.DMA,
        ],
    )
    def sc_edge(ei_hbm, asrc_hbm, adst_hbm, xp_hbm, msp_hbm, zero_hbm,
                z_hbm, asrc_v, adst_v, xp_v, ibufA, ibufB, didxA, didxB,
                payA, payB, m_v, z_sh, gsA, gsB, ssA, ssB):
        c = lax.axis_index("c")
        s = lax.axis_index("s")
        wid = s * NC + c
        base = wid * ept
        pltpu.sync_copy(asrc_hbm, asrc_v)
        pltpu.sync_copy(adst_hbm, adst_v)
        pltpu.sync_copy(xp_hbm, xp_v)
        pltpu.sync_copy(msp_hbm, m_v)
        pltpu.sync_copy(zero_hbm, z_sh.at[pl.ds(s * rows_per, rows_per)])
        plsc.subcore_barrier()

        m_vecs = [m_v[pl.ds(h * 16, 16)] for h in range(hh)]
        lane = lax.iota(jnp.int32, 16)
        cols = [jnp.full((16,), col, jnp.int32) for col in range(ZW)]
        offs = [jnp.full((16,), h * np1, jnp.int32) for h in range(hh)]

        def fetch(j, ibuf, gs):
            off = base + j * CH
            pltpu.async_copy(ei_hbm.at[0, pl.ds(off, CH)], ibuf.at[0], gs)
            pltpu.async_copy(ei_hbm.at[1, pl.ds(off, CH)], ibuf.at[1], gs)

        def fetch_wait(ibuf, gs):
            pltpu.make_async_copy(ei_hbm.at[0, pl.ds(0, CH)], ibuf.at[0],
                                  gs).wait()
            pltpu.make_async_copy(ei_hbm.at[0, pl.ds(0, CH)], ibuf.at[1],
                                  gs).wait()

        def compute_group(g, ibuf, didx, pay_v):
            row16 = lane + (g * 16)
            src16 = ibuf[0, pl.ds(g * 16, 16)]
            dst16 = ibuf[1, pl.ds(g * 16, 16)]
            didx[0, pl.ds(g * 16, 16)] = dst16
            a_s = [plsc.load_gather(asrc_v, [src16 + offs[h]])
                   for h in range(hh)]
            ad01r = plsc.load_gather(adst_v, [dst16])
            ad23r = plsc.load_gather(adst_v, [dst16 + offs[1]])
            ad0, ad1 = plsc.unpack(plsc.bitcast(ad01r, jnp.bfloat16),
                                   format=plsc.PackFormat.INTERLEAVED,
                                   preferred_element_type=jnp.float32)
            ad2, ad3 = plsc.unpack(plsc.bitcast(ad23r, jnp.bfloat16),
                                   format=plsc.PackFormat.INTERLEAVED,
                                   preferred_element_type=jnp.float32)
            a_d = [ad0, ad1, ad2, ad3]
            x01r = plsc.load_gather(xp_v, [src16])
            x23r = plsc.load_gather(xp_v, [src16 + offs[1]])
            x4r = plsc.load_gather(xp_v, [src16 + offs[2]])
            x0, x1 = plsc.unpack(plsc.bitcast(x01r, jnp.bfloat16),
                                 format=plsc.PackFormat.INTERLEAVED,
                                 preferred_element_type=jnp.float32)
            x2, x3 = plsc.unpack(plsc.bitcast(x23r, jnp.bfloat16),
                                 format=plsc.PackFormat.INTERLEAVED,
                                 preferred_element_type=jnp.float32)
            xs = [x0, x1, x2, x3, plsc.bitcast(x4r, jnp.float32)]
            for h in range(hh):
                al = a_s[h] + a_d[h]
                al = jnp.where(al > 0.0, al, 0.2 * al)
                p = jnp.exp(al - m_vecs[h])
                plsc.store_scatter(pay_v, [row16, cols[hh * in_dim + h]], p)
                for i in range(in_dim):
                    plsc.store_scatter(pay_v, [row16, cols[h * in_dim + i]],
                                       p * xs[i])

        fetch(0, ibufA, gsA)
        fetch(1, ibufB, gsB)

        def body(j2, carry):
            for par, (ibuf, didx, pay, gs, ss) in enumerate(
                    [(ibufA, didxA, payA, gsA, ssA),
                     (ibufB, didxB, payB, gsB, ssB)]):
                j = j2 * 2 + par

                @pl.when(j2 > 0)
                def _():
                    pltpu.make_async_copy(z_hbm.at[0, pl.ds(0, CH)], pay,
                                          ss).wait()

                fetch_wait(ibuf, gs)
                for g in range(CH // 16):
                    compute_group(g, ibuf, didx, pay)

                @pl.when(j + 2 < kfull)
                def _():
                    fetch(j + 2, ibuf, gs)

                pltpu.async_copy(pay, z_sh.at[didx.at[0]], ss, add=True)
            return carry

        lax.fori_loop(0, kfull // 2, body, 0)
        pltpu.make_async_copy(z_hbm.at[0, pl.ds(0, CH)], payA, ssA).wait()
        pltpu.make_async_copy(z_hbm.at[0, pl.ds(0, CH)], payB, ssB).wait()

        if tail:
            off = base + kfull * CH
            pltpu.sync_copy(ei_hbm.at[0, pl.ds(off, tail)],
                            ibufA.at[0, pl.ds(0, tail)])
            pltpu.sync_copy(ei_hbm.at[1, pl.ds(off, tail)],
                            ibufA.at[1, pl.ds(0, tail)])
            junk = jnp.full((16,), n, jnp.int32)
            for g in range(tail // 16, CH // 16):
                didxA[0, pl.ds(g * 16, 16)] = junk
            for g in range(tail // 16):
                compute_group(g, ibufA, didxA, payA)
            pltpu.sync_copy(payA, z_sh.at[didxA.at[0]], add=True)

        plsc.subcore_barrier()
        pltpu.sync_copy(z_sh.at[pl.ds(s * rows_per, rows_per)],
                        z_hbm.at[c, pl.ds(s * rows_per, rows_per)])

    return sc_edge


def _epilogue_body(hh, cc, in_dim, act, z0_ref, z1_ref, xt_ref, ast_ref,
                   adt_ref, m_ref, wg_ref, bg_ref, w1_ref, b1_ref, w2_ref,
                   b2_ref, w3_ref, b3_ref, mean_ref, std_ref):
    zz = z0_ref[...] + z1_ref[...]
    al = ast_ref[...] + adt_ref[...]
    al = jnp.where(al > 0.0, al, 0.2 * al)
    p = jnp.exp(al - m_ref[...])
    xv = xt_ref[...]
    wg = wg_ref[...]
    embs = []
    for h in range(hh):
        zh = zz[:, h * in_dim:(h + 1) * in_dim] + p[:, h:h + 1] * xv
        den = zz[:, hh * in_dim + h:hh * in_dim + h + 1] + p[:, h:h + 1]
        zh = zh / (den + 1e-16)
        embs.append(jnp.dot(zh, wg[:, h * cc:(h + 1) * cc],
                            preferred_element_type=jnp.float32))
    emb = jnp.concatenate(embs, axis=1) + bg_ref[...]
    hdn = jnp.maximum(
        jnp.dot(emb, w1_ref[...], preferred_element_type=jnp.float32)
        + b1_ref[...], 0.0)
    hdn = jnp.maximum(
        jnp.dot(hdn, w2_ref[...], preferred_element_type=jnp.float32)
        + b2_ref[...], 0.0)
    a_out = jnp.dot(hdn, w3_ref[...], preferred_element_type=jnp.float32) \
        + b3_ref[...]
    mean_ref[...] = a_out[:, :act]
    log_std = jnp.clip(a_out[:, act:], -20.0, 2.0)
    std_ref[...] = jnp.exp(log_std)


def _epilogue(z0, z1, xt, ast, adt, m, W_gat, b_gat, aW1, ab1, aW2, ab2,
              aW3, ab3):
    nt, in_dim = xt.shape
    hh = ast.shape[1]
    cc = W_gat.shape[1] // hh
    hid = aW1.shape[1]
    act = aW3.shape[1] // 2
    blk = 1000
    grid = nt // blk
    c0 = lambda i: (0, 0)
    return pl.pallas_call(
        functools.partial(_epilogue_body, hh, cc, in_dim, act),
        grid=(grid,),
        in_specs=[
            pl.BlockSpec((blk, ZW), lambda i: (i, 0)),
            pl.BlockSpec((blk, ZW), lambda i: (i, 0)),
            pl.BlockSpec((blk, in_dim), lambda i: (i, 0)),
            pl.BlockSpec((blk, hh), lambda i: (i, 0)),
            pl.BlockSpec((blk, hh), lambda i: (i, 0)),
            pl.BlockSpec((1, hh), c0),
            pl.BlockSpec((in_dim, hh * cc), c0),
            pl.BlockSpec((1, hh * cc), c0),
            pl.BlockSpec((hh * cc, hid), c0),
            pl.BlockSpec((1, hid), c0),
            pl.BlockSpec((hid, hid), c0),
            pl.BlockSpec((1, hid), c0),
            pl.BlockSpec((hid, 2 * act), c0),
            pl.BlockSpec((1, 2 * act), c0),
        ],
        out_specs=[
            pl.BlockSpec((blk, act), lambda i: (i, 0)),
            pl.BlockSpec((blk, act), lambda i: (i, 0)),
        ],
        out_shape=[
            jax.ShapeDtypeStruct((nt, act), jnp.float32),
            jax.ShapeDtypeStruct((nt, act), jnp.float32),
        ],
    )(z0, z1, xt, ast, adt, m, W_gat, b_gat.reshape(1, -1), aW1,
      ab1.reshape(1, -1), aW2, ab2.reshape(1, -1), aW3, ab3.reshape(1, -1))


def _critic_body(obs_ref, w1_ref, b1_ref, w2_ref, b2_ref, w3_ref, b3_ref,
                 out_ref):
    v = jnp.maximum(
        jnp.dot(obs_ref[...], w1_ref[...], preferred_element_type=jnp.float32)
        + b1_ref[...], 0.0)
    v = jnp.maximum(
        jnp.dot(v, w2_ref[...], preferred_element_type=jnp.float32)
        + b2_ref[...], 0.0)
    out_ref[...] = jnp.dot(v, w3_ref[...],
                           preferred_element_type=jnp.float32) + b3_ref[...]


def _critic(obs, cW1, cb1, cW2, cb2, cW3, cb3):
    b, in_dim = obs.shape
    hid = cW1.shape[1]
    return pl.pallas_call(
        _critic_body,
        out_shape=jax.ShapeDtypeStruct((b, 1), jnp.float32),
    )(obs, cW1, cb1.reshape(1, -1), cW2, cb2.reshape(1, -1), cW3,
      cb3.reshape(1, -1))


def kernel(x, edge_index, observation, W_gat, att_src, att_dst, b_gat,
           aW1, ab1, aW2, ab2, aW3, ab3, cW1, cb1, cW2, cb2, cW3, cb3):
    n, in_dim = x.shape
    hh, cc = att_src.shape
    e = edge_index.shape[1]

    a_src, a_dst, m = _prologue(x, W_gat, att_src, att_dst)

    ept = e // NW
    assert e % NW == 0

    nrow = (n + NS + 127) // 128 * 128
    np1 = -(-(n + 1) // 8) * 8
    pad1 = lambda a: jnp.pad(a, (0, np1 - n))
    asrc_flat = jnp.concatenate([pad1(a_src[:, h]) for h in range(hh)])
    pair = lambda a, b: (
        jax.lax.bitcast_convert_type(a.astype(jnp.bfloat16),
                                     jnp.uint16).astype(jnp.uint32)
        | (jax.lax.bitcast_convert_type(b.astype(jnp.bfloat16),
                                        jnp.uint16).astype(jnp.uint32) << 16)
    ).astype(jnp.int32)
    adst_flat = jnp.concatenate([pad1(pair(a_dst[:, 0], a_dst[:, 1])),
                                 pad1(pair(a_dst[:, 2], a_dst[:, 3]))])
    x4 = jax.lax.bitcast_convert_type(x[:, 4], jnp.int32)
    xp = jnp.concatenate([pad1(pair(x[:, 0], x[:, 1])),
                          pad1(pair(x[:, 2], x[:, 3])), pad1(x4)])
    msp = jnp.broadcast_to(m.reshape(hh, 1), (hh, 16)).reshape(-1)
    zeros = jnp.zeros((nrow // NS, ZW), jnp.float32)

    sc_edge = _make_sc_edge(n, in_dim, hh, ept, nrow, np1)
    z = sc_edge(edge_index.astype(jnp.int32), asrc_flat, adst_flat, xp,
                msp, zeros)

    ng = n // TEAM_MOD
    zt = z[:, :n, :].reshape(NC, ng, TEAM_MOD, ZW)[:, :, :TEAM_KEEP, :]
    zt = zt.reshape(NC, ng * TEAM_KEEP, ZW)
    sel = lambda a: a.reshape(ng, TEAM_MOD, -1)[:, :TEAM_KEEP].reshape(
        ng * TEAM_KEEP, -1)
    xt, ast, adt = sel(x), sel(a_src), sel(a_dst)

    mean, std = _epilogue(zt[0], zt[1], xt, ast, adt, m, W_gat, b_gat,
                          aW1, ab1, aW2, ab2, aW3, ab3)
    value = _critic(observation, cW1, cb1, cW2, cb2, cW3, cb3)
    return (mean, std, value)


# speedup vs baseline: 162.3524x; 1.1657x over previous
import functools

import jax
import jax.numpy as jnp
from jax import lax
from jax.experimental import pallas as pl
from jax.experimental.pallas import tpu as pltpu
from jax.experimental.pallas import tpu_sc as plsc

NC = 2
NS = 16
NW = NC * NS
CH = 128
ZW = 32
TEAM_MOD = 10
TEAM_KEEP = 5


def _prologue_body(H, C, x_ref, w_ref, ats_ref, atd_ref,
                   asrc_ref, adst_ref, m_ref):
    wg = w_ref[...]
    ws = jnp.concatenate(
        [jnp.sum(wg[:, h * C:(h + 1) * C] * ats_ref[h:h + 1, :], axis=1,
                 keepdims=True) for h in range(H)], axis=1)
    wd = jnp.concatenate(
        [jnp.sum(wg[:, h * C:(h + 1) * C] * atd_ref[h:h + 1, :], axis=1,
                 keepdims=True) for h in range(H)], axis=1)
    a_s = jnp.dot(x_ref[...], ws, preferred_element_type=jnp.float32)
    a_d = jnp.dot(x_ref[...], wd, preferred_element_type=jnp.float32)
    asrc_ref[...] = a_s
    adst_ref[...] = a_d
    t = (jnp.max(a_s, axis=0, keepdims=True)
         + jnp.max(a_d, axis=0, keepdims=True))
    m_ref[...] = jnp.where(t > 0.0, t, 0.2 * t)


def _prologue(x, W_gat, att_src, att_dst):
    n, in_dim = x.shape
    hh, cc = att_src.shape
    return pl.pallas_call(
        functools.partial(_prologue_body, hh, cc),
        out_shape=[
            jax.ShapeDtypeStruct((n, hh), jnp.float32),
            jax.ShapeDtypeStruct((n, hh), jnp.float32),
            jax.ShapeDtypeStruct((1, hh), jnp.float32),
        ],
    )(x, W_gat, att_src, att_dst)


def _make_sc_edge(n, in_dim, hh, ept, nrow, np1, tmod, tkeep):
    mesh = plsc.VectorSubcoreMesh(core_axis_name="c", subcore_axis_name="s",
                                  num_cores=NC, num_subcores=NS)
    rows_per = nrow // NS
    kfull = ept // CH
    assert tmod == 10 and n < 52429
    n_team = (n // tmod) * tkeep

    tail = ept % CH
    assert kfull % 2 == 0 and tail % 16 == 0 and tail < CH

    @functools.partial(
        pl.kernel,
        out_type=jax.ShapeDtypeStruct((NC, nrow, ZW), jnp.float32),
        mesh=mesh,
        compiler_params=pltpu.CompilerParams(use_tc_tiling_on_sc=False,
                                             needs_layout_passes=False),
        scratch_types=[
            pltpu.VMEM((hh * np1,), jnp.float32),
            pltpu.VMEM(((hh // 2) * np1,), jnp.int32),
            pltpu.VMEM((3 * np1,), jnp.int32),
            pltpu.VMEM((2, CH), jnp.int32),
            pltpu.VMEM((2, CH), jnp.int32),
            pltpu.VMEM((1, CH), jnp.int32),
            pltpu.VMEM((1, CH), jnp.int32),
            pltpu.VMEM((CH, ZW), jnp.float32),
            pltpu.VMEM((CH, ZW), jnp.float32),
            pltpu.VMEM((hh * 16,), jnp.float32),
            pltpu.VMEM_SHARED((nrow, ZW), jnp.float32),
            pltpu.SemaphoreType.DMA,
            pltpu.SemaphoreType.DMA,
            pltpu.SemaphoreType.DMA,
            pltpu.SemaphoreType.DMA,
        ],
    )
    def sc_edge(ei_hbm, asrc_hbm, adst_hbm, xp_hbm, msp_hbm, zero_hbm,
                z_hbm, asrc_v, adst_v, xp_v, ibufA, ibufB, didxA, didxB,
                payA, payB, m_v, z_sh, gsA, gsB, ssA, ssB):
        c = lax.axis_index("c")
        s = lax.axis_index("s")
        wid = s * NC + c
        base = wid * ept
        pltpu.sync_copy(asrc_hbm, asrc_v)
        pltpu.sync_copy(adst_hbm, adst_v)
        pltpu.sync_copy(xp_hbm, xp_v)
        pltpu.sync_copy(msp_hbm, m_v)
        pltpu.sync_copy(zero_hbm, z_sh.at[pl.ds(s * rows_per, rows_per)])
        plsc.subcore_barrier()

        m_vecs = [m_v[pl.ds(h * 16, 16)] for h in range(hh)]
        lane = lax.iota(jnp.int32, 16)
        cols = [jnp.full((16,), col, jnp.int32) for col in range(ZW)]
        offs = [jnp.full((16,), h * np1, jnp.int32) for h in range(hh)]

        def fetch(j, ibuf, gs):
            off = base + j * CH
            pltpu.async_copy(ei_hbm.at[0, pl.ds(off, CH)], ibuf.at[0], gs)
            pltpu.async_copy(ei_hbm.at[1, pl.ds(off, CH)], ibuf.at[1], gs)

        def fetch_wait(ibuf, gs):
            pltpu.make_async_copy(ei_hbm.at[0, pl.ds(0, CH)], ibuf.at[0],
                                  gs).wait()
            pltpu.make_async_copy(ei_hbm.at[0, pl.ds(0, CH)], ibuf.at[1],
                                  gs).wait()

        def compute_group(g, ibuf, didx, pay_v):
            row16 = lane + (g * 16)
            src16 = ibuf[0, pl.ds(g * 16, 16)]
            dst16 = ibuf[1, pl.ds(g * 16, 16)]
            q = jax.lax.shift_right_logical(dst16 * 52429, 19)
            r = dst16 - q * tmod
            didx[0, pl.ds(g * 16, 16)] = jnp.where(
                r < tkeep, q * tkeep + r, jnp.full((16,), n_team, jnp.int32))
            a_s = [plsc.load_gather(asrc_v, [src16 + offs[h]])
                   for h in range(hh)]
            ad01r = plsc.load_gather(adst_v, [dst16])
            ad23r = plsc.load_gather(adst_v, [dst16 + offs[1]])
            ad0, ad1 = plsc.unpack(plsc.bitcast(ad01r, jnp.bfloat16),
                                   format=plsc.PackFormat.INTERLEAVED,
                                   preferred_element_type=jnp.float32)
            ad2, ad3 = plsc.unpack(plsc.bitcast(ad23r, jnp.bfloat16),
                                   format=plsc.PackFormat.INTERLEAVED,
                                   preferred_element_type=jnp.float32)
            a_d = [ad0, ad1, ad2, ad3]
            x01r = plsc.load_gather(xp_v, [src16])
            x23r = plsc.load_gather(xp_v, [src16 + offs[1]])
            x4r = plsc.load_gather(xp_v, [src16 + offs[2]])
            x0, x1 = plsc.unpack(plsc.bitcast(x01r, jnp.bfloat16),
                                 format=plsc.PackFormat.INTERLEAVED,
                                 preferred_element_type=jnp.float32)
            x2, x3 = plsc.unpack(plsc.bitcast(x23r, jnp.bfloat16),
                                 format=plsc.PackFormat.INTERLEAVED,
                                 preferred_element_type=jnp.float32)
            xs = [x0, x1, x2, x3, plsc.bitcast(x4r, jnp.float32)]
            for h in range(hh):
                al = a_s[h] + a_d[h]
                al = jnp.where(al > 0.0, al, 0.2 * al)
                p = jnp.exp(al - m_vecs[h])
                plsc.store_scatter(pay_v, [row16, cols[hh * in_dim + h]], p)
                for i in range(in_dim):
                    plsc.store_scatter(pay_v, [row16, cols[h * in_dim + i]],
                                       p * xs[i])

        fetch(0, ibufA, gsA)
        fetch(1, ibufB, gsB)

        def body(j2, carry):
            for par, (ibuf, didx, pay, gs, ss) in enumerate(
                    [(ibufA, didxA, payA, gsA, ssA),
                     (ibufB, didxB, payB, gsB, ssB)]):
                j = j2 * 2 + par

                @pl.when(j2 > 0)
                def _():
                    pltpu.make_async_copy(z_hbm.at[0, pl.ds(0, CH)], pay,
                                          ss).wait()

                fetch_wait(ibuf, gs)
                for g in range(CH // 16):
                    compute_group(g, ibuf, didx, pay)

                @pl.when(j + 2 < kfull)
                def _():
                    fetch(j + 2, ibuf, gs)

                pltpu.async_copy(pay, z_sh.at[didx.at[0]], ss, add=True)
            return carry

        lax.fori_loop(0, kfull // 2, body, 0)
        pltpu.make_async_copy(z_hbm.at[0, pl.ds(0, CH)], payA, ssA).wait()
        pltpu.make_async_copy(z_hbm.at[0, pl.ds(0, CH)], payB, ssB).wait()

        if tail:
            off = base + kfull * CH
            pltpu.sync_copy(ei_hbm.at[0, pl.ds(off, tail)],
                            ibufA.at[0, pl.ds(0, tail)])
            pltpu.sync_copy(ei_hbm.at[1, pl.ds(off, tail)],
                            ibufA.at[1, pl.ds(0, tail)])
            junk = jnp.full((16,), n_team, jnp.int32)
            for g in range(tail // 16, CH // 16):
                didxA[0, pl.ds(g * 16, 16)] = junk
            for g in range(tail // 16):
                compute_group(g, ibufA, didxA, payA)
            pltpu.sync_copy(payA, z_sh.at[didxA.at[0]], add=True)

        plsc.subcore_barrier()
        pltpu.sync_copy(z_sh.at[pl.ds(s * rows_per, rows_per)],
                        z_hbm.at[c, pl.ds(s * rows_per, rows_per)])

    return sc_edge


def _epilogue_body(hh, cc, in_dim, act, z0_ref, z1_ref, xt_ref, ast_ref,
                   adt_ref, m_ref, wg_ref, bg_ref, w1_ref, b1_ref, w2_ref,
                   b2_ref, w3_ref, b3_ref, mean_ref, std_ref):
    zz = z0_ref[...] + z1_ref[...]
    al = ast_ref[...] + adt_ref[...]
    al = jnp.where(al > 0.0, al, 0.2 * al)
    p = jnp.exp(al - m_ref[...])
    xv = xt_ref[...]
    wg = wg_ref[...]
    embs = []
    for h in range(hh):
        zh = zz[:, h * in_dim:(h + 1) * in_dim] + p[:, h:h + 1] * xv
        den = zz[:, hh * in_dim + h:hh * in_dim + h + 1] + p[:, h:h + 1]
        zh = zh / (den + 1e-16)
        embs.append(jnp.dot(zh, wg[:, h * cc:(h + 1) * cc],
                            preferred_element_type=jnp.float32))
    emb = jnp.concatenate(embs, axis=1) + bg_ref[...]
    hdn = jnp.maximum(
        jnp.dot(emb, w1_ref[...], preferred_element_type=jnp.float32)
        + b1_ref[...], 0.0)
    hdn = jnp.maximum(
        jnp.dot(hdn, w2_ref[...], preferred_element_type=jnp.float32)
        + b2_ref[...], 0.0)
    a_out = jnp.dot(hdn, w3_ref[...], preferred_element_type=jnp.float32) \
        + b3_ref[...]
    mean_ref[...] = a_out[:, :act]
    log_std = jnp.clip(a_out[:, act:], -20.0, 2.0)
    std_ref[...] = jnp.exp(log_std)


def _epilogue(z0, z1, xt, ast, adt, m, W_gat, b_gat, aW1, ab1, aW2, ab2,
              aW3, ab3):
    nt, in_dim = xt.shape
    hh = ast.shape[1]
    cc = W_gat.shape[1] // hh
    hid = aW1.shape[1]
    act = aW3.shape[1] // 2
    blk = 1000
    grid = nt // blk
    c0 = lambda i: (0, 0)
    return pl.pallas_call(
        functools.partial(_epilogue_body, hh, cc, in_dim, act),
        grid=(grid,),
        in_specs=[
            pl.BlockSpec((blk, ZW), lambda i: (i, 0)),
            pl.BlockSpec((blk, ZW), lambda i: (i, 0)),
            pl.BlockSpec((blk, in_dim), lambda i: (i, 0)),
            pl.BlockSpec((blk, hh), lambda i: (i, 0)),
            pl.BlockSpec((blk, hh), lambda i: (i, 0)),
            pl.BlockSpec((1, hh), c0),
            pl.BlockSpec((in_dim, hh * cc), c0),
            pl.BlockSpec((1, hh * cc), c0),
            pl.BlockSpec((hh * cc, hid), c0),
            pl.BlockSpec((1, hid), c0),
            pl.BlockSpec((hid, hid), c0),
            pl.BlockSpec((1, hid), c0),
            pl.BlockSpec((hid, 2 * act), c0),
            pl.BlockSpec((1, 2 * act), c0),
        ],
        out_specs=[
            pl.BlockSpec((blk, act), lambda i: (i, 0)),
            pl.BlockSpec((blk, act), lambda i: (i, 0)),
        ],
        out_shape=[
            jax.ShapeDtypeStruct((nt, act), jnp.float32),
            jax.ShapeDtypeStruct((nt, act), jnp.float32),
        ],
    )(z0, z1, xt, ast, adt, m, W_gat, b_gat.reshape(1, -1), aW1,
      ab1.reshape(1, -1), aW2, ab2.reshape(1, -1), aW3, ab3.reshape(1, -1))


def _critic_body(obs_ref, w1_ref, b1_ref, w2_ref, b2_ref, w3_ref, b3_ref,
                 out_ref):
    v = jnp.maximum(
        jnp.dot(obs_ref[...], w1_ref[...], preferred_element_type=jnp.float32)
        + b1_ref[...], 0.0)
    v = jnp.maximum(
        jnp.dot(v, w2_ref[...], preferred_element_type=jnp.float32)
        + b2_ref[...], 0.0)
    out_ref[...] = jnp.dot(v, w3_ref[...],
                           preferred_element_type=jnp.float32) + b3_ref[...]


def _critic(obs, cW1, cb1, cW2, cb2, cW3, cb3):
    b, in_dim = obs.shape
    hid = cW1.shape[1]
    return pl.pallas_call(
        _critic_body,
        out_shape=jax.ShapeDtypeStruct((b, 1), jnp.float32),
    )(obs, cW1, cb1.reshape(1, -1), cW2, cb2.reshape(1, -1), cW3,
      cb3.reshape(1, -1))


def kernel(x, edge_index, observation, W_gat, att_src, att_dst, b_gat,
           aW1, ab1, aW2, ab2, aW3, ab3, cW1, cb1, cW2, cb2, cW3, cb3):
    n, in_dim = x.shape
    hh, cc = att_src.shape
    e = edge_index.shape[1]

    a_src, a_dst, m = _prologue(x, W_gat, att_src, att_dst)

    ept = e // NW
    assert e % NW == 0

    n_team = (n // TEAM_MOD) * TEAM_KEEP
    nrow = (n_team + NS + 127) // 128 * 128
    np1 = -(-(n + 1) // 8) * 8
    pad1 = lambda a: jnp.pad(a, (0, np1 - n))
    asrc_flat = jnp.concatenate([pad1(a_src[:, h]) for h in range(hh)])
    pair = lambda a, b: (
        jax.lax.bitcast_convert_type(a.astype(jnp.bfloat16),
                                     jnp.uint16).astype(jnp.uint32)
        | (jax.lax.bitcast_convert_type(b.astype(jnp.bfloat16),
                                        jnp.uint16).astype(jnp.uint32) << 16)
    ).astype(jnp.int32)
    adst_flat = jnp.concatenate([pad1(pair(a_dst[:, 0], a_dst[:, 1])),
                                 pad1(pair(a_dst[:, 2], a_dst[:, 3]))])
    x4 = jax.lax.bitcast_convert_type(x[:, 4], jnp.int32)
    xp = jnp.concatenate([pad1(pair(x[:, 0], x[:, 1])),
                          pad1(pair(x[:, 2], x[:, 3])), pad1(x4)])
    msp = jnp.broadcast_to(m.reshape(hh, 1), (hh, 16)).reshape(-1)
    zeros = jnp.zeros((nrow // NS, ZW), jnp.float32)

    sc_edge = _make_sc_edge(n, in_dim, hh, ept, nrow, np1,
                            TEAM_MOD, TEAM_KEEP)
    z = sc_edge(edge_index.astype(jnp.int32), asrc_flat, adst_flat, xp,
                msp, zeros)

    ng = n // TEAM_MOD
    zt = z[:, :n_team, :]
    sel = lambda a: a.reshape(ng, TEAM_MOD, -1)[:, :TEAM_KEEP].reshape(
        ng * TEAM_KEEP, -1)
    xt, ast, adt = sel(x), sel(a_src), sel(a_dst)

    mean, std = _epilogue(zt[0], zt[1], xt, ast, adt, m, W_gat, b_gat,
                          aW1, ab1, aW2, ab2, aW3, ab3)
    value = _critic(observation, cW1, cb1, cW2, cb2, cW3, cb3)
    return (mean, std, value)
